# Initial kernel scaffold; baseline (speedup 1.0000x reference)
import jax
import jax.numpy as jnp
from jax.experimental import pallas as pl


def kernel(inputs):
    raise NotImplementedError("write your pallas kernel here")


# speedup vs baseline: 3.9090x; 3.9090x over previous
import functools

import jax
import jax.numpy as jnp
from jax.experimental import pallas as pl
from jax.experimental.pallas import tpu as pltpu

_BATCH = 16384
_N = 8
_NSETS = 254
_STEPS = 48
_BIG = 3.0e38
_KSM = 1.0 / (1.0 - float(jnp.exp(-1.0)))


_CH = 16
_CHROWS = _BATCH // _CH


def _extract_body(x_ref, out_ref, v_ref):
    pid = pl.program_id(0)
    lane = jax.lax.broadcasted_iota(jnp.int32, (1, 128), 1)
    midx = lane + 128 * pid + 1

    def build_chunk(ch, _):
        x = x_ref[pl.ds(ch * _CHROWS, _CHROWS), :]
        base = jnp.sum(x * x, axis=1, keepdims=True)
        v = jnp.broadcast_to(base, (_CHROWS, 128))
        for i in range(_N):
            bit = ((midx >> i) & 1).astype(jnp.float32)
            v = v + (1.0 - 2.0 * x[:, i : i + 1]) * bit
        v_ref[pl.ds(ch * _CHROWS, _CHROWS), :] = v
        return 0

    jax.lax.fori_loop(0, _CH, build_chunk, 0)

    def body(_, carry):
        r, acc = carry

        def minc(ch, m):
            vv = v_ref[pl.ds(ch * _CHROWS, _CHROWS), :]
            return jnp.minimum(m, jnp.min(vv, axis=0, keepdims=True))

        m = jax.lax.fori_loop(
            0, _CH, minc, jnp.full((1, 128), _BIG, jnp.float32)
        )

        def maskc(ch, c):
            vv = v_ref[pl.ds(ch * _CHROWS, _CHROWS), :]
            eq = vv == m
            v_ref[pl.ds(ch * _CHROWS, _CHROWS), :] = jnp.where(eq, _BIG, vv)
            return c + jnp.sum(eq.astype(jnp.float32), axis=0, keepdims=True)

        c = jax.lax.fori_loop(0, _CH, maskc, jnp.zeros((1, 128), jnp.float32))
        rn = r + c
        acc = acc + m * (jnp.exp(-r) - jnp.exp(-rn))
        return rn, acc

    r0 = jnp.zeros((1, 128), jnp.float32)
    _, acc = jax.lax.fori_loop(0, _STEPS, body, (r0, r0))
    out_ref[...] = (acc * _KSM)[None]


def _finish_body(acc_ref, out_ref):
    acc = acc_ref[...]
    lane = jax.lax.broadcasted_iota(jnp.int32, (2, 1, 128), 2)
    row = jax.lax.broadcasted_iota(jnp.int32, (2, 1, 128), 0)
    valid = (lane + 128 * row) < _NSETS
    masked = jnp.where(valid, acc, 0.0)[:, 0, :]
    total = jnp.sum(masked, axis=(0, 1), keepdims=True)
    out_ref[...] = total / float(_NSETS * _BATCH)


@jax.jit
def kernel(inputs):
    acc = pl.pallas_call(
        _extract_body,
        grid=(2,),
        in_specs=[pl.BlockSpec((_BATCH, _N), lambda i: (0, 0))],
        out_specs=pl.BlockSpec((1, 1, 128), lambda i: (i, 0, 0)),
        out_shape=jax.ShapeDtypeStruct((2, 1, 128), jnp.float32),
        scratch_shapes=[pltpu.VMEM((_BATCH, 128), jnp.float32)],
        compiler_params=pltpu.CompilerParams(
            dimension_semantics=("parallel",)
        ),
    )(inputs)
    out = pl.pallas_call(
        _finish_body,
        out_shape=jax.ShapeDtypeStruct((1, 1), jnp.float32),
    )(acc)
    return out[0, 0]


# speedup vs baseline: 25.5472x; 6.5355x over previous
import jax
import jax.numpy as jnp
from jax.experimental import pallas as pl
from jax.experimental.pallas import tpu as pltpu

_BATCH = 16384
_N = 8
_NSETS = 254
_RUN = 64
_NRUNS = _BATCH // 8 // _RUN


def _flip(z, axis):
    n = z.shape[axis]
    idx = [slice(None)] * axis
    return jnp.concatenate(
        [z[tuple(idx + [slice(i, i + 1)])] for i in range(n - 1, -1, -1)],
        axis=axis,
    )


def _bitonic_merge(z, m):
    d = m // 2
    while d >= 1:
        s = z.shape[1:]
        zz = z.reshape((m // (2 * d), 2, d) + s)
        lo = jnp.minimum(zz[:, 0], zz[:, 1])
        hi = jnp.maximum(zz[:, 0], zz[:, 1])
        z = jnp.stack([lo, hi], axis=1).reshape((m,) + s)
        d //= 2
    return z


def _sort_runs(v):
    ln = 1
    while ln < _RUN:
        g = _RUN // (2 * ln)
        y = v.reshape((g, 2, ln, 8, 128))
        a = y[:, 0]
        b = _flip(y[:, 1], 1)
        z = jnp.concatenate([a, b], axis=1)
        d = ln
        m = 2 * ln
        while d >= 1:
            zz = z.reshape((g, m // (2 * d), 2, d, 8, 128))
            lo = jnp.minimum(zz[:, :, 0], zz[:, :, 1])
            hi = jnp.maximum(zz[:, :, 0], zz[:, :, 1])
            z = jnp.stack([lo, hi], axis=2).reshape((g, m, 8, 128))
            d //= 2
        v = z.reshape((_RUN, 8, 128))
        ln *= 2
    return v


def _prune_merge(a, b):
    z = jnp.minimum(a, _flip(b, 0))
    return _bitonic_merge(z, a.shape[0])


def _select_body(x_ref, out_ref, w_ref):
    pid = pl.program_id(0)
    lane = jax.lax.broadcasted_iota(jnp.int32, (1, 128), 1)
    midx = lane + 128 * pid + 1

    rows = 8 * _RUN

    def build_chunk(ch, _):
        x = x_ref[pl.ds(ch * rows, rows), :]
        base = jnp.sum(x * x, axis=1, keepdims=True)
        v = jnp.broadcast_to(base, (rows, 128))
        for i in range(_N):
            bit = ((midx >> i) & 1).astype(jnp.float32)
            v = v + (1.0 - 2.0 * x[:, i : i + 1]) * bit
        v = _sort_runs(v.reshape(_RUN, 8, 128))
        w_ref[pl.ds(ch * _RUN, _RUN), :, :] = v
        return 0

    jax.lax.fori_loop(0, _NRUNS, build_chunk, 0)

    nruns = _NRUNS
    while nruns > 1:

        def merge_pair(j, _):
            z = w_ref[pl.ds(j * 2 * _RUN, 2 * _RUN), :, :]
            m = _prune_merge(z[:_RUN], z[_RUN:])
            w_ref[pl.ds(j * _RUN, _RUN), :, :] = m
            return 0

        jax.lax.fori_loop(0, nruns // 2, merge_pair, 0)
        nruns //= 2

    f = w_ref[0:_RUN, :, :]
    f = _prune_merge(f[:, 0:4, :], f[:, 4:8, :])
    f = _prune_merge(f[:, 0:2, :], f[:, 2:4, :])
    f = _prune_merge(f[:, 0:1, :], f[:, 1:2, :])
    k = jax.lax.broadcasted_iota(jnp.int32, (_RUN, 1, 1), 0)
    wts = jnp.exp(-k.astype(jnp.float32))
    acc = jnp.sum(f * wts, axis=0)
    out_ref[...] = acc[None]


def _finish_body(acc_ref, out_ref):
    acc = acc_ref[...]
    lane = jax.lax.broadcasted_iota(jnp.int32, (2, 1, 128), 2)
    row = jax.lax.broadcasted_iota(jnp.int32, (2, 1, 128), 0)
    valid = (lane + 128 * row) < _NSETS
    masked = jnp.where(valid, acc, 0.0)[:, 0, :]
    total = jnp.sum(masked, axis=(0, 1), keepdims=True)
    out_ref[...] = total / float(_NSETS * _BATCH)


@jax.jit
def kernel(inputs):
    acc = pl.pallas_call(
        _select_body,
        grid=(2,),
        in_specs=[pl.BlockSpec((_BATCH, _N), lambda i: (0, 0))],
        out_specs=pl.BlockSpec((1, 1, 128), lambda i: (i, 0, 0)),
        out_shape=jax.ShapeDtypeStruct((2, 1, 128), jnp.float32),
        scratch_shapes=[pltpu.VMEM((_BATCH // 8, 8, 128), jnp.float32)],
        compiler_params=pltpu.CompilerParams(
            dimension_semantics=("parallel",)
        ),
    )(inputs)
    out = pl.pallas_call(
        _finish_body,
        out_shape=jax.ShapeDtypeStruct((1, 1), jnp.float32),
    )(acc)
    return out[0, 0]


# speedup vs baseline: 37.5977x; 1.4717x over previous
import jax
import jax.numpy as jnp
from jax.experimental import pallas as pl
from jax.experimental.pallas import tpu as pltpu

_BATCH = 16384
_N = 8
_NSETS = 254
_RUN = 64
_NRUNS = _BATCH // 8 // _RUN


def _flip(z, axis):
    n = z.shape[axis]
    idx = [slice(None)] * axis
    return jnp.concatenate(
        [z[tuple(idx + [slice(i, i + 1)])] for i in range(n - 1, -1, -1)],
        axis=axis,
    )


def _bitonic_merge(z, m):
    d = m // 2
    while d >= 1:
        s = z.shape[1:]
        zz = z.reshape((m // (2 * d), 2, d) + s)
        lo = jnp.minimum(zz[:, 0], zz[:, 1])
        hi = jnp.maximum(zz[:, 0], zz[:, 1])
        z = jnp.stack([lo, hi], axis=1).reshape((m,) + s)
        d //= 2
    return z


def _sort_runs(v):
    ln = 1
    while ln < _RUN:
        g = _RUN // (2 * ln)
        y = v.reshape((g, 2, ln, 8, 128))
        a = y[:, 0]
        b = _flip(y[:, 1], 1)
        z = jnp.concatenate([a, b], axis=1)
        d = ln
        m = 2 * ln
        while d >= 1:
            zz = z.reshape((g, m // (2 * d), 2, d, 8, 128))
            lo = jnp.minimum(zz[:, :, 0], zz[:, :, 1])
            hi = jnp.maximum(zz[:, :, 0], zz[:, :, 1])
            z = jnp.stack([lo, hi], axis=2).reshape((g, m, 8, 128))
            d //= 2
        v = z.reshape((_RUN, 8, 128))
        ln *= 2
    return v


def _prune_merge(a, b):
    z = jnp.minimum(a, _flip(b, 0))
    return _bitonic_merge(z, a.shape[0])


def _select_body(x_ref, out_ref, w_ref):
    pid = pl.program_id(0)
    lane = jax.lax.broadcasted_iota(jnp.int32, (_N, 128), 1)
    bitpos = jax.lax.broadcasted_iota(jnp.int32, (_N, 128), 0)
    midx = lane + 128 * pid + 1
    amat = ((midx >> bitpos) & 1).astype(jnp.float32)

    rows = 8 * _RUN

    def build_chunk(ch, _):
        x = x_ref[pl.ds(ch * rows, rows), :]
        base = jnp.sum(x * x, axis=1, keepdims=True)
        d = 1.0 - 2.0 * x
        v = base + jax.lax.dot(
            d, amat, precision=jax.lax.Precision.HIGHEST,
            preferred_element_type=jnp.float32,
        )
        v = _sort_runs(v.reshape(_RUN, 8, 128))
        w_ref[pl.ds(ch * _RUN, _RUN), :, :] = v
        return 0

    jax.lax.fori_loop(0, _NRUNS, build_chunk, 0)

    nruns = _NRUNS
    while nruns > 1:

        def merge_pair(j, _):
            z = w_ref[pl.ds(j * 2 * _RUN, 2 * _RUN), :, :]
            m = _prune_merge(z[:_RUN], z[_RUN:])
            w_ref[pl.ds(j * _RUN, _RUN), :, :] = m
            return 0

        jax.lax.fori_loop(0, nruns // 2, merge_pair, 0)
        nruns //= 2

    f = w_ref[0:_RUN, :, :]
    f = _prune_merge(f[:, 0:4, :], f[:, 4:8, :])
    f = _prune_merge(f[:, 0:2, :], f[:, 2:4, :])
    f = _prune_merge(f[:, 0:1, :], f[:, 1:2, :])
    k = jax.lax.broadcasted_iota(jnp.int32, (_RUN, 1, 1), 0)
    wts = jnp.exp(-k.astype(jnp.float32))
    acc = jnp.sum(f * wts, axis=0)
    out_ref[...] = acc[None]


def _finish_body(acc_ref, out_ref):
    acc = acc_ref[...]
    lane = jax.lax.broadcasted_iota(jnp.int32, (2, 1, 128), 2)
    row = jax.lax.broadcasted_iota(jnp.int32, (2, 1, 128), 0)
    valid = (lane + 128 * row) < _NSETS
    masked = jnp.where(valid, acc, 0.0)[:, 0, :]
    total = jnp.sum(masked, axis=(0, 1), keepdims=True)
    out_ref[...] = total / float(_NSETS * _BATCH)


@jax.jit
def kernel(inputs):
    acc = pl.pallas_call(
        _select_body,
        grid=(2,),
        in_specs=[pl.BlockSpec((_BATCH, _N), lambda i: (0, 0))],
        out_specs=pl.BlockSpec((1, 1, 128), lambda i: (i, 0, 0)),
        out_shape=jax.ShapeDtypeStruct((2, 1, 128), jnp.float32),
        scratch_shapes=[pltpu.VMEM((_BATCH // 8, 8, 128), jnp.float32)],
        compiler_params=pltpu.CompilerParams(
            dimension_semantics=("parallel",)
        ),
    )(inputs)
    out = pl.pallas_call(
        _finish_body,
        out_shape=jax.ShapeDtypeStruct((1, 1), jnp.float32),
    )(acc)
    return out[0, 0]


# speedup vs baseline: 38.4891x; 1.0237x over previous
import jax
import jax.numpy as jnp
from jax.experimental import pallas as pl
from jax.experimental.pallas import tpu as pltpu

_BATCH = 16384
_N = 8
_NSETS = 254
_RUN = 64
_NRUNS = _BATCH // 8 // _RUN
_L = 256


def _flip(z, axis):
    n = z.shape[axis]
    idx = [slice(None)] * axis
    return jnp.concatenate(
        [z[tuple(idx + [slice(i, i + 1)])] for i in range(n - 1, -1, -1)],
        axis=axis,
    )


def _bitonic_merge(z, m):
    d = m // 2
    while d >= 1:
        s = z.shape[1:]
        zz = z.reshape((m // (2 * d), 2, d) + s)
        lo = jnp.minimum(zz[:, 0], zz[:, 1])
        hi = jnp.maximum(zz[:, 0], zz[:, 1])
        z = jnp.stack([lo, hi], axis=1).reshape((m,) + s)
        d //= 2
    return z


def _sort_runs(v):
    ln = 1
    while ln < _RUN:
        g = _RUN // (2 * ln)
        y = v.reshape((g, 2, ln, 8, _L))
        a = y[:, 0]
        b = _flip(y[:, 1], 1)
        z = jnp.concatenate([a, b], axis=1)
        d = ln
        m = 2 * ln
        while d >= 1:
            zz = z.reshape((g, m // (2 * d), 2, d, 8, _L))
            lo = jnp.minimum(zz[:, :, 0], zz[:, :, 1])
            hi = jnp.maximum(zz[:, :, 0], zz[:, :, 1])
            z = jnp.stack([lo, hi], axis=2).reshape((g, m, 8, _L))
            d //= 2
        v = z.reshape((_RUN, 8, _L))
        ln *= 2
    return v


def _prune_merge(a, b):
    z = jnp.minimum(a, _flip(b, 0))
    return _bitonic_merge(z, a.shape[0])


def _select_body(x_ref, out_ref, w_ref):
    lane = jax.lax.broadcasted_iota(jnp.int32, (_N, _L), 1)
    bitpos = jax.lax.broadcasted_iota(jnp.int32, (_N, _L), 0)
    midx = lane + 1
    amat = ((midx >> bitpos) & 1).astype(jnp.float32)

    rows = 8 * _RUN

    def build_chunk(ch, _):
        x = x_ref[pl.ds(ch * rows, rows), :]
        base = jnp.sum(x * x, axis=1, keepdims=True)
        d = 1.0 - 2.0 * x
        v = base + jax.lax.dot(
            d, amat, precision=jax.lax.Precision.HIGHEST,
            preferred_element_type=jnp.float32,
        )
        v = _sort_runs(v.reshape(_RUN, 8, _L))
        w_ref[pl.ds(ch * _RUN, _RUN), :, :] = v
        return 0

    jax.lax.fori_loop(0, _NRUNS, build_chunk, 0)

    nruns = _NRUNS
    while nruns > 1:

        def merge_pair(j, _):
            z = w_ref[pl.ds(j * 2 * _RUN, 2 * _RUN), :, :]
            m = _prune_merge(z[:_RUN], z[_RUN:])
            w_ref[pl.ds(j * _RUN, _RUN), :, :] = m
            return 0

        jax.lax.fori_loop(0, nruns // 2, merge_pair, 0)
        nruns //= 2

    f = w_ref[0:_RUN, :, :]
    f = _prune_merge(f[:, 0:4, :], f[:, 4:8, :])
    f = _prune_merge(f[:, 0:2, :], f[:, 2:4, :])
    f = _prune_merge(f[:, 0:1, :], f[:, 1:2, :])
    k = jax.lax.broadcasted_iota(jnp.int32, (_RUN, 1, 1), 0)
    wts = jnp.exp(-k.astype(jnp.float32))
    acc = jnp.sum(f * wts, axis=0)
    out_ref[...] = acc[None]


def _finish_body(acc_ref, out_ref):
    acc = acc_ref[...]
    lane = jax.lax.broadcasted_iota(jnp.int32, (1, 1, _L), 2)
    valid = lane < _NSETS
    masked = jnp.where(valid, acc, 0.0)[:, 0, :]
    total = jnp.sum(masked, axis=(0, 1), keepdims=True)
    out_ref[...] = total / float(_NSETS * _BATCH)


@jax.jit
def kernel(inputs):
    acc = pl.pallas_call(
        _select_body,
        out_shape=jax.ShapeDtypeStruct((1, 1, _L), jnp.float32),
        scratch_shapes=[pltpu.VMEM((_BATCH // 8, 8, _L), jnp.float32)],
    )(inputs)
    out = pl.pallas_call(
        _finish_body,
        out_shape=jax.ShapeDtypeStruct((1, 1), jnp.float32),
    )(acc)
    return out[0, 0]


# speedup vs baseline: 51.6572x; 1.3421x over previous
import jax
import jax.numpy as jnp
from jax.experimental import pallas as pl
from jax.experimental.pallas import tpu as pltpu

_BATCH = 16384
_N = 8
_NSETS = 254
_RUN = 64
_SUBL = 16
_DEPTH = _BATCH // _SUBL
_NRUNS = _DEPTH // _RUN
_L = 256


def _flip(z, axis):
    n = z.shape[axis]
    idx = [slice(None)] * axis
    return jnp.concatenate(
        [z[tuple(idx + [slice(i, i + 1)])] for i in range(n - 1, -1, -1)],
        axis=axis,
    )


def _bitonic_merge(z, m):
    d = m // 2
    while d >= 1:
        s = z.shape[1:]
        zz = z.reshape((m // (2 * d), 2, d) + s)
        lo = jnp.minimum(zz[:, 0], zz[:, 1])
        hi = jnp.maximum(zz[:, 0], zz[:, 1])
        z = jnp.stack([lo, hi], axis=1).reshape((m,) + s)
        d //= 2
    return z


def _sort_runs(v):
    s = v.shape[1]
    ln = 1
    while ln < _RUN:
        g = _RUN // (2 * ln)
        y = v.reshape((g, 2, ln, s, _L))
        a = y[:, 0]
        b = _flip(y[:, 1], 1)
        z = jnp.concatenate([a, b], axis=1)
        d = ln
        m = 2 * ln
        while d >= 1:
            zz = z.reshape((g, m // (2 * d), 2, d, s, _L))
            lo = jnp.minimum(zz[:, :, 0], zz[:, :, 1])
            hi = jnp.maximum(zz[:, :, 0], zz[:, :, 1])
            z = jnp.stack([lo, hi], axis=2).reshape((g, m, s, _L))
            d //= 2
        v = z.reshape((_RUN, s, _L))
        ln *= 2
    return v


def _prune_merge(a, b):
    z = jnp.minimum(a, _flip(b, 0))
    return _bitonic_merge(z, a.shape[0])


def _select_body(x_ref, out_ref, w_ref):
    lane = jax.lax.broadcasted_iota(jnp.int32, (_N, _L), 1)
    bitpos = jax.lax.broadcasted_iota(jnp.int32, (_N, _L), 0)
    midx = lane + 1
    amat = ((midx >> bitpos) & 1).astype(jnp.float32)

    rows = _SUBL * _RUN

    def build_chunk(ch, _):
        x = x_ref[pl.ds(ch * rows, rows), :]
        base = jnp.sum(x * x, axis=1, keepdims=True)
        d = 1.0 - 2.0 * x
        v = base + jax.lax.dot(
            d, amat, precision=jax.lax.Precision.HIGHEST,
            preferred_element_type=jnp.float32,
        )
        v = _sort_runs(v.astype(jnp.bfloat16).reshape(_RUN, _SUBL, _L))
        w_ref[pl.ds(ch * _RUN, _RUN), :, :] = v
        return 0

    jax.lax.fori_loop(0, _NRUNS, build_chunk, 0)

    nruns = _NRUNS
    while nruns > 1:

        def merge_pair(j, _):
            z = w_ref[pl.ds(j * 2 * _RUN, 2 * _RUN), :, :]
            m = _prune_merge(z[:_RUN], z[_RUN:])
            w_ref[pl.ds(j * _RUN, _RUN), :, :] = m
            return 0

        jax.lax.fori_loop(0, nruns // 2, merge_pair, 0)
        nruns //= 2

    f = w_ref[0:_RUN, :, :].astype(jnp.float32)
    f = _prune_merge(f[:, 0:8, :], f[:, 8:16, :])
    f = _prune_merge(f[:, 0:4, :], f[:, 4:8, :])
    f = _prune_merge(f[:, 0:2, :], f[:, 2:4, :])
    f = _prune_merge(f[:, 0:1, :], f[:, 1:2, :])
    k = jax.lax.broadcasted_iota(jnp.int32, (_RUN, 1, 1), 0)
    wts = jnp.exp(-k.astype(jnp.float32))
    acc = jnp.sum(f * wts, axis=0)
    out_ref[...] = acc[None]


def _finish_body(acc_ref, out_ref):
    acc = acc_ref[...]
    lane = jax.lax.broadcasted_iota(jnp.int32, (1, 1, _L), 2)
    valid = lane < _NSETS
    masked = jnp.where(valid, acc, 0.0)[:, 0, :]
    total = jnp.sum(masked, axis=(0, 1), keepdims=True)
    out_ref[...] = total / float(_NSETS * _BATCH)


@jax.jit
def kernel(inputs):
    acc = pl.pallas_call(
        _select_body,
        out_shape=jax.ShapeDtypeStruct((1, 1, _L), jnp.float32),
        scratch_shapes=[pltpu.VMEM((_DEPTH, _SUBL, _L), jnp.bfloat16)],
    )(inputs)
    out = pl.pallas_call(
        _finish_body,
        out_shape=jax.ShapeDtypeStruct((1, 1), jnp.float32),
    )(acc)
    return out[0, 0]


# speedup vs baseline: 74.6185x; 1.4445x over previous
import jax
import jax.numpy as jnp
from jax.experimental import pallas as pl
from jax.experimental.pallas import tpu as pltpu

_BATCH = 16384
_N = 8
_NSETS = 254
_RUN = 64
_SUBL = 16
_DEPTH = _BATCH // _SUBL
_NRUNS = _DEPTH // _RUN
_L = 256


def _flip(z, axis):
    n = z.shape[axis]
    idx = [slice(None)] * axis
    return jnp.concatenate(
        [z[tuple(idx + [slice(i, i + 1)])] for i in range(n - 1, -1, -1)],
        axis=axis,
    )


def _bitonic_merge(z, m):
    d = m // 2
    while d >= 1:
        s = z.shape[1:]
        zz = z.reshape((m // (2 * d), 2, d) + s)
        lo = jnp.minimum(zz[:, 0], zz[:, 1])
        hi = jnp.maximum(zz[:, 0], zz[:, 1])
        z = jnp.stack([lo, hi], axis=1).reshape((m,) + s)
        d //= 2
    return z


def _sort_runs(v):
    s = v.shape[1]
    ln = 1
    while ln < _RUN:
        g = _RUN // (2 * ln)
        y = v.reshape((g, 2, ln, s, _L))
        a = y[:, 0]
        b = _flip(y[:, 1], 1)
        z = jnp.concatenate([a, b], axis=1)
        d = ln
        m = 2 * ln
        while d >= 1:
            zz = z.reshape((g, m // (2 * d), 2, d, s, _L))
            lo = jnp.minimum(zz[:, :, 0], zz[:, :, 1])
            hi = jnp.maximum(zz[:, :, 0], zz[:, :, 1])
            z = jnp.stack([lo, hi], axis=2).reshape((g, m, s, _L))
            d //= 2
        v = z.reshape((_RUN, s, _L))
        ln *= 2
    return v


def _prune_merge(a, b):
    z = jnp.minimum(a, _flip(b, 0))
    return _bitonic_merge(z, a.shape[0])


def _select_body(x_ref, out_ref, w_ref):
    lane = jax.lax.broadcasted_iota(jnp.int32, (_N, _L), 1)
    bitpos = jax.lax.broadcasted_iota(jnp.int32, (_N, _L), 0)
    midx = lane + 1
    amat = ((midx >> bitpos) & 1).astype(jnp.float32)

    rows = _SUBL * _RUN

    def build_chunk(ch, _):
        x = x_ref[pl.ds(ch * rows, rows), :]
        base = jnp.sum(x * x, axis=1, keepdims=True)
        d = 1.0 - 2.0 * x
        v = base + jax.lax.dot(
            d, amat, preferred_element_type=jnp.float32,
        )
        v = _sort_runs(v.astype(jnp.bfloat16).reshape(_RUN, _SUBL, _L))
        w_ref[pl.ds(ch * _RUN, _RUN), :, :] = v
        return 0

    jax.lax.fori_loop(0, _NRUNS, build_chunk, 0)

    nruns = _NRUNS
    while nruns > 1:

        def merge_pair(j, _):
            z = w_ref[pl.ds(j * 2 * _RUN, 2 * _RUN), :, :]
            m = _prune_merge(z[:_RUN], z[_RUN:])
            w_ref[pl.ds(j * _RUN, _RUN), :, :] = m
            return 0

        jax.lax.fori_loop(0, nruns // 2, merge_pair, 0)
        nruns //= 2

    f = w_ref[0:_RUN, :, :].astype(jnp.float32)
    f = _prune_merge(f[:, 0:8, :], f[:, 8:16, :])
    f = _prune_merge(f[:, 0:4, :], f[:, 4:8, :])
    f = _prune_merge(f[:, 0:2, :], f[:, 2:4, :])
    f = _prune_merge(f[:, 0:1, :], f[:, 1:2, :])
    k = jax.lax.broadcasted_iota(jnp.int32, (_RUN, 1, 1), 0)
    wts = jnp.exp(-k.astype(jnp.float32))
    acc = jnp.sum(f * wts, axis=0)
    out_ref[...] = acc[None]


def _finish_body(acc_ref, out_ref):
    acc = acc_ref[...]
    lane = jax.lax.broadcasted_iota(jnp.int32, (1, 1, _L), 2)
    valid = lane < _NSETS
    masked = jnp.where(valid, acc, 0.0)[:, 0, :]
    total = jnp.sum(masked, axis=(0, 1), keepdims=True)
    out_ref[...] = total / float(_NSETS * _BATCH)


@jax.jit
def kernel(inputs):
    acc = pl.pallas_call(
        _select_body,
        out_shape=jax.ShapeDtypeStruct((1, 1, _L), jnp.float32),
        scratch_shapes=[pltpu.VMEM((_DEPTH, _SUBL, _L), jnp.bfloat16)],
    )(inputs)
    out = pl.pallas_call(
        _finish_body,
        out_shape=jax.ShapeDtypeStruct((1, 1), jnp.float32),
    )(acc)
    return out[0, 0]


# speedup vs baseline: 77.3938x; 1.0372x over previous
import jax
import jax.numpy as jnp
from jax.experimental import pallas as pl
from jax.experimental.pallas import tpu as pltpu

_BATCH = 16384
_N = 8
_NSETS = 254
_RUN = 64
_SUBL = 16
_DEPTH = _BATCH // _SUBL
_NRUNS = _DEPTH // _RUN
_L = 256


def _flip(z, axis):
    n = z.shape[axis]
    idx = [slice(None)] * axis
    return jnp.concatenate(
        [z[tuple(idx + [slice(i, i + 1)])] for i in range(n - 1, -1, -1)],
        axis=axis,
    )


def _bitonic_merge(z, m):
    d = m // 2
    while d >= 1:
        s = z.shape[1:]
        zz = z.reshape((m // (2 * d), 2, d) + s)
        lo = jnp.minimum(zz[:, 0], zz[:, 1])
        hi = jnp.maximum(zz[:, 0], zz[:, 1])
        z = jnp.stack([lo, hi], axis=1).reshape((m,) + s)
        d //= 2
    return z


def _sort_runs(v):
    s = v.shape[1]
    ln = 1
    while ln < _RUN:
        g = _RUN // (2 * ln)
        y = v.reshape((g, 2, ln, s, _L))
        a = y[:, 0]
        b = _flip(y[:, 1], 1)
        z = jnp.concatenate([a, b], axis=1)
        d = ln
        m = 2 * ln
        while d >= 1:
            zz = z.reshape((g, m // (2 * d), 2, d, s, _L))
            lo = jnp.minimum(zz[:, :, 0], zz[:, :, 1])
            hi = jnp.maximum(zz[:, :, 0], zz[:, :, 1])
            z = jnp.stack([lo, hi], axis=2).reshape((g, m, s, _L))
            d //= 2
        v = z.reshape((_RUN, s, _L))
        ln *= 2
    return v


def _prune_merge(a, b):
    z = jnp.minimum(a, _flip(b, 0))
    return _bitonic_merge(z, a.shape[0])


def _select_body(x_ref, out_ref, w_ref):
    lane = jax.lax.broadcasted_iota(jnp.int32, (_N, _L), 1)
    bitpos = jax.lax.broadcasted_iota(jnp.int32, (_N, _L), 0)
    midx = lane + 1
    amat = ((midx >> bitpos) & 1).astype(jnp.float32)

    rows = 2 * _SUBL * _RUN

    def build_chunk(ch, _):
        x = x_ref[pl.ds(ch * rows, rows), :]
        base = jnp.sum(x * x, axis=1, keepdims=True)
        d = 1.0 - 2.0 * x
        v = base + jax.lax.dot(
            d, amat, preferred_element_type=jnp.float32,
        )
        v = v.astype(jnp.bfloat16).reshape(2 * _RUN, _SUBL, _L)
        va = _sort_runs(v[:_RUN])
        vb = _sort_runs(v[_RUN:])
        w_ref[pl.ds(ch * _RUN, _RUN), :, :] = _prune_merge(va, vb)
        return 0

    jax.lax.fori_loop(0, _NRUNS // 2, build_chunk, 0)

    nruns = _NRUNS // 2
    while nruns > 1:

        def merge_pair(j, _):
            z = w_ref[pl.ds(j * 2 * _RUN, 2 * _RUN), :, :]
            m = _prune_merge(z[:_RUN], z[_RUN:])
            w_ref[pl.ds(j * _RUN, _RUN), :, :] = m
            return 0

        jax.lax.fori_loop(0, nruns // 2, merge_pair, 0)
        nruns //= 2

    f = w_ref[0:_RUN, :, :].astype(jnp.float32)
    f = _prune_merge(f[:, 0:8, :], f[:, 8:16, :])
    f = _prune_merge(f[:, 0:4, :], f[:, 4:8, :])
    f = _prune_merge(f[:, 0:2, :], f[:, 2:4, :])
    f = _prune_merge(f[:, 0:1, :], f[:, 1:2, :])
    k = jax.lax.broadcasted_iota(jnp.int32, (_RUN, 1, 1), 0)
    wts = jnp.exp(-k.astype(jnp.float32))
    acc = jnp.sum(f * wts, axis=0)
    out_ref[...] = acc[None]


def _finish_body(acc_ref, out_ref):
    acc = acc_ref[...]
    lane = jax.lax.broadcasted_iota(jnp.int32, (1, 1, _L), 2)
    valid = lane < _NSETS
    masked = jnp.where(valid, acc, 0.0)[:, 0, :]
    total = jnp.sum(masked, axis=(0, 1), keepdims=True)
    out_ref[...] = total / float(_NSETS * _BATCH)


@jax.jit
def kernel(inputs):
    acc = pl.pallas_call(
        _select_body,
        out_shape=jax.ShapeDtypeStruct((1, 1, _L), jnp.float32),
        scratch_shapes=[pltpu.VMEM((_DEPTH, _SUBL, _L), jnp.bfloat16)],
    )(inputs)
    out = pl.pallas_call(
        _finish_body,
        out_shape=jax.ShapeDtypeStruct((1, 1), jnp.float32),
    )(acc)
    return out[0, 0]


# speedup vs baseline: 80.4927x; 1.0400x over previous
import jax
import jax.numpy as jnp
from jax.experimental import pallas as pl
from jax.experimental.pallas import tpu as pltpu

_BATCH = 16384
_N = 8
_NSETS = 254
_RUN = 64
_SUBL = 16
_DEPTH = _BATCH // _SUBL
_NRUNS = _DEPTH // _RUN
_L = 256


def _flip(z, axis):
    n = z.shape[axis]
    idx = [slice(None)] * axis
    return jnp.concatenate(
        [z[tuple(idx + [slice(i, i + 1)])] for i in range(n - 1, -1, -1)],
        axis=axis,
    )


def _bitonic_merge(z, m):
    d = m // 2
    while d >= 1:
        s = z.shape[1:]
        zz = z.reshape((m // (2 * d), 2, d) + s)
        lo = jnp.minimum(zz[:, 0], zz[:, 1])
        hi = jnp.maximum(zz[:, 0], zz[:, 1])
        z = jnp.stack([lo, hi], axis=1).reshape((m,) + s)
        d //= 2
    return z


def _sort_runs(v):
    s = v.shape[1]
    ln = 1
    while ln < _RUN:
        g = _RUN // (2 * ln)
        y = v.reshape((g, 2, ln, s, _L))
        a = y[:, 0]
        b = _flip(y[:, 1], 1)
        z = jnp.concatenate([a, b], axis=1)
        d = ln
        m = 2 * ln
        while d >= 1:
            zz = z.reshape((g, m // (2 * d), 2, d, s, _L))
            lo = jnp.minimum(zz[:, :, 0], zz[:, :, 1])
            hi = jnp.maximum(zz[:, :, 0], zz[:, :, 1])
            z = jnp.stack([lo, hi], axis=2).reshape((g, m, s, _L))
            d //= 2
        v = z.reshape((_RUN, s, _L))
        ln *= 2
    return v


def _prune_merge(a, b):
    z = jnp.minimum(a, _flip(b, 0))
    return _bitonic_merge(z, a.shape[0])


def _select_body(x_ref, out_ref, w_ref):
    lane = jax.lax.broadcasted_iota(jnp.int32, (_N, _L), 1)
    bitpos = jax.lax.broadcasted_iota(jnp.int32, (_N, _L), 0)
    midx = lane + 1
    amat = ((midx >> bitpos) & 1).astype(jnp.float32)

    rows = 2 * _SUBL * _RUN

    def build_chunk(ch, _):
        x = x_ref[pl.ds(ch * rows, rows), :]
        base = jnp.sum(x * x, axis=1, keepdims=True)
        d = 1.0 - 2.0 * x
        v = base + jax.lax.dot(
            d, amat, preferred_element_type=jnp.float32,
        )
        v = v.astype(jnp.bfloat16).reshape(2 * _RUN, _SUBL, _L)
        va = _sort_runs(v[:_RUN])
        vb = _sort_runs(v[_RUN:])
        w_ref[pl.ds(ch * _RUN, _RUN), :, :] = _prune_merge(va, vb)
        return 0

    jax.lax.fori_loop(0, _NRUNS // 2, build_chunk, 0)

    nruns = _NRUNS // 2
    while nruns > 1:

        def merge_pair(j, _):
            z = w_ref[pl.ds(j * 2 * _RUN, 2 * _RUN), :, :]
            m = _prune_merge(z[:_RUN], z[_RUN:])
            w_ref[pl.ds(j * _RUN, _RUN), :, :] = m
            return 0

        jax.lax.fori_loop(0, nruns // 2, merge_pair, 0)
        nruns //= 2

    f = w_ref[0:_RUN, :, :].astype(jnp.float32)
    f = _prune_merge(f[:, 0:8, :], f[:, 8:16, :])
    f = _prune_merge(f[:, 0:4, :], f[:, 4:8, :])
    f = _prune_merge(f[:, 0:2, :], f[:, 2:4, :])
    f = _prune_merge(f[:, 0:1, :], f[:, 1:2, :])
    k = jax.lax.broadcasted_iota(jnp.int32, (_RUN, 1, 1), 0)
    wts = jnp.exp(-k.astype(jnp.float32))
    acc = jnp.sum(f * wts, axis=0)
    lanei = jax.lax.broadcasted_iota(jnp.int32, (1, _L), 1)
    masked = jnp.where(lanei < _NSETS, acc, 0.0)
    total = jnp.sum(masked, axis=(0, 1), keepdims=True)
    out_ref[...] = total / float(_NSETS * _BATCH)


@jax.jit
def kernel(inputs):
    out = pl.pallas_call(
        _select_body,
        out_shape=jax.ShapeDtypeStruct((1, 1), jnp.float32),
        scratch_shapes=[pltpu.VMEM((_DEPTH, _SUBL, _L), jnp.bfloat16)],
    )(inputs)
    return out[0, 0]


# speedup vs baseline: 81.0283x; 1.0067x over previous
import jax
import jax.numpy as jnp
from jax.experimental import pallas as pl
from jax.experimental.pallas import tpu as pltpu

_BATCH = 16384
_N = 8
_NSETS = 254
_RUN = 64
_SUBL = 16
_DEPTH = _BATCH // _SUBL
_NRUNS = _DEPTH // _RUN
_L = 256


def _flip(z, axis):
    n = z.shape[axis]
    idx = [slice(None)] * axis
    return jnp.concatenate(
        [z[tuple(idx + [slice(i, i + 1)])] for i in range(n - 1, -1, -1)],
        axis=axis,
    )


def _bitonic_merge(z, m):
    d = m // 2
    while d >= 1:
        s = z.shape[1:]
        zz = z.reshape((m // (2 * d), 2, d) + s)
        lo = jnp.minimum(zz[:, 0], zz[:, 1])
        hi = jnp.maximum(zz[:, 0], zz[:, 1])
        z = jnp.stack([lo, hi], axis=1).reshape((m,) + s)
        d //= 2
    return z


def _sort_runs(v):
    s = v.shape[1]
    ln = 1
    while ln < _RUN:
        g = _RUN // (2 * ln)
        y = v.reshape((g, 2, ln, s, _L))
        a = y[:, 0]
        b = _flip(y[:, 1], 1)
        z = jnp.concatenate([a, b], axis=1)
        d = ln
        m = 2 * ln
        while d >= 1:
            zz = z.reshape((g, m // (2 * d), 2, d, s, _L))
            lo = jnp.minimum(zz[:, :, 0], zz[:, :, 1])
            hi = jnp.maximum(zz[:, :, 0], zz[:, :, 1])
            z = jnp.stack([lo, hi], axis=2).reshape((g, m, s, _L))
            d //= 2
        v = z.reshape((_RUN, s, _L))
        ln *= 2
    return v


def _prune_merge(a, b):
    z = jnp.minimum(a, _flip(b, 0))
    return _bitonic_merge(z, a.shape[0])


def _select_body(x_ref, out_ref, w_ref):
    lane = jax.lax.broadcasted_iota(jnp.int32, (_N, _L), 1)
    bitpos = jax.lax.broadcasted_iota(jnp.int32, (_N, _L), 0)
    midx = lane + 1
    amat = ((midx >> bitpos) & 1).astype(jnp.float32)

    rows = 4 * _SUBL * _RUN

    def build_chunk(ch, _):
        x = x_ref[pl.ds(ch * rows, rows), :]
        base = jnp.sum(x * x, axis=1, keepdims=True)
        d = 1.0 - 2.0 * x
        v = base + jax.lax.dot(
            d, amat, preferred_element_type=jnp.float32,
        )
        v = v.astype(jnp.bfloat16).reshape(4 * _RUN, _SUBL, _L)
        runs = [_sort_runs(v[i * _RUN : (i + 1) * _RUN]) for i in range(4)]
        m = _prune_merge(
            _prune_merge(runs[0], runs[1]), _prune_merge(runs[2], runs[3])
        )
        w_ref[pl.ds(ch * _RUN, _RUN), :, :] = m
        return 0

    jax.lax.fori_loop(0, _NRUNS // 4, build_chunk, 0)

    nruns = _NRUNS // 4
    while nruns > 1:

        def merge_pair(j, _):
            z = w_ref[pl.ds(j * 2 * _RUN, 2 * _RUN), :, :]
            m = _prune_merge(z[:_RUN], z[_RUN:])
            w_ref[pl.ds(j * _RUN, _RUN), :, :] = m
            return 0

        jax.lax.fori_loop(0, nruns // 2, merge_pair, 0)
        nruns //= 2

    f = w_ref[0:_RUN, :, :].astype(jnp.float32)
    f = _prune_merge(f[:, 0:8, :], f[:, 8:16, :])
    f = _prune_merge(f[:, 0:4, :], f[:, 4:8, :])
    f = _prune_merge(f[:, 0:2, :], f[:, 2:4, :])
    f = _prune_merge(f[:, 0:1, :], f[:, 1:2, :])
    k = jax.lax.broadcasted_iota(jnp.int32, (_RUN, 1, 1), 0)
    wts = jnp.exp(-k.astype(jnp.float32))
    acc = jnp.sum(f * wts, axis=0)
    lanei = jax.lax.broadcasted_iota(jnp.int32, (1, _L), 1)
    masked = jnp.where(lanei < _NSETS, acc, 0.0)
    total = jnp.sum(masked, axis=(0, 1), keepdims=True)
    out_ref[...] = total / float(_NSETS * _BATCH)


@jax.jit
def kernel(inputs):
    out = pl.pallas_call(
        _select_body,
        out_shape=jax.ShapeDtypeStruct((1, 1), jnp.float32),
        scratch_shapes=[pltpu.VMEM((_DEPTH, _SUBL, _L), jnp.bfloat16)],
    )(inputs)
    return out[0, 0]
